# baseline (speedup 1.0000x reference)
import functools

import jax
import jax.numpy as jnp
from jax import lax
from jax.experimental import pallas as pl
from jax.experimental.pallas import tpu as pltpu
from jax.experimental.pallas import tpu_sc as plsc

N_NODES = 10000
D = 128
N_EDGES = 320000

NC = 2
NS = 16
NW = NC * NS

NPAD = 10112
ROWS_PER_TILE = NPAD // NS

CH = 128
NCHUNK = 79
EW = CH * NCHUNK
EPAD = EW * NW
NCHUNKS_TOTAL = EPAD // CH
CPP = 2 * NCHUNK
CN0 = CPP
CN1 = CPP - CN0
_ZSIZES = [CH] * (ROWS_PER_TILE // CH) + (
    [ROWS_PER_TILE % CH] if ROWS_PER_TILE % CH else [])

_MESH = plsc.VectorSubcoreMesh(core_axis_name="c", subcore_axis_name="s")


@functools.partial(
    pl.kernel,
    out_type=jax.ShapeDtypeStruct((NC, NPAD, D), jnp.float32),
    mesh=_MESH,
    scratch_types=[
        pltpu.VMEM((2, CH), jnp.int32),
        pltpu.VMEM((CH, D), jnp.float32),
        pltpu.VMEM_SHARED((NPAD, D), jnp.float32),
        pltpu.SemaphoreType.DMA,
    ],
)
def _deg_kernel(onesz_hbm, dst_hbm, out_hbm, dst_v, ones_v, acc, sem_idx):
    c = lax.axis_index("c")
    s = lax.axis_index("s")
    base = (s * NC + c) * NCHUNK

    pltpu.sync_copy(onesz_hbm.at[1], ones_v)
    off = 0
    for sz in _ZSIZES:
        pltpu.sync_copy(ones_v.at[pl.ds(0, sz)],
                        acc.at[pl.ds(s * ROWS_PER_TILE + off, sz)])
        off += sz
    pltpu.sync_copy(onesz_hbm.at[0], ones_v)
    plsc.subcore_barrier()

    pltpu.sync_copy(dst_hbm.at[base], dst_v.at[0])
    pltpu.make_async_copy(dst_hbm.at[base + 1], dst_v.at[1], sem_idx).start()

    def step(j, _):
        b = lax.rem(j, 2)
        pltpu.sync_copy(ones_v, acc.at[dst_v.at[b]], add=True)

        @pl.when(j + 1 < NCHUNK)
        def _():
            pltpu.make_async_copy(dst_hbm.at[base + j + 1],
                                  dst_v.at[1 - b], sem_idx).wait()

        @pl.when(j + 2 < NCHUNK)
        def _():
            pltpu.make_async_copy(dst_hbm.at[base + j + 2],
                                  dst_v.at[b], sem_idx).start()
        return 0
    lax.fori_loop(0, NCHUNK, step, 0)
    plsc.subcore_barrier()

    pltpu.sync_copy(acc.at[pl.ds(s * ROWS_PER_TILE, ROWS_PER_TILE)],
                    out_hbm.at[c, pl.ds(s * ROWS_PER_TILE, ROWS_PER_TILE)])


@functools.partial(
    pl.kernel,
    out_type=jax.ShapeDtypeStruct((NC, NPAD, D), jnp.float32),
    mesh=_MESH,
    scratch_types=[
        pltpu.VMEM((3, CH), jnp.int32),
        pltpu.VMEM((3, CH), jnp.int32),
        pltpu.VMEM((3, CH, D), jnp.float32),
        pltpu.VMEM_SHARED((NPAD, D), jnp.float32),
        pltpu.SemaphoreType.DMA,
        pltpu.SemaphoreType.DMA,
    ],
)
def _scatter_kernel(g_hbm, src_hbm, dst_hbm, out_hbm,
                    src_v, dst_v, rows_v, acc, sem, sem_idx):
    c = lax.axis_index("c")
    s = lax.axis_index("s")
    base = s * CPP + c * CN0
    cnt = jnp.where(c == 0, CN0, CN1)

    zero16 = jnp.zeros((16,), jnp.float32)

    def fill(r, _):
        def fcol(k, _):
            rows_v[0, r, pl.ds(k * 16, 16)] = zero16
            return 0
        return lax.fori_loop(0, D // 16, fcol, 0)
    lax.fori_loop(0, CH, fill, 0)

    off = 0
    for sz in _ZSIZES:
        pltpu.sync_copy(rows_v.at[0, pl.ds(0, sz)],
                        acc.at[pl.ds(s * ROWS_PER_TILE + off, sz)])
        off += sz
    plsc.subcore_barrier()

    @pl.when(0 < cnt)
    def _():
        pltpu.sync_copy(src_hbm.at[base], src_v.at[0])
        pltpu.sync_copy(dst_hbm.at[base], dst_v.at[0])
        pltpu.make_async_copy(g_hbm.at[src_v.at[0]], rows_v.at[0],
                              sem).start()

    @pl.when(1 < cnt)
    def _():
        pltpu.sync_copy(src_hbm.at[base + 1], src_v.at[1])
        pltpu.sync_copy(dst_hbm.at[base + 1], dst_v.at[1])
        pltpu.make_async_copy(g_hbm.at[src_v.at[1]], rows_v.at[1],
                              sem).start()

    @pl.when(2 < cnt)
    def _():
        pltpu.make_async_copy(src_hbm.at[base + 2], src_v.at[2],
                              sem_idx).start()
        pltpu.make_async_copy(dst_hbm.at[base + 2], dst_v.at[2],
                              sem_idx).start()

    def step(j, _):
        b = lax.rem(j, 3)
        b2 = lax.rem(j + 2, 3)
        pltpu.make_async_copy(g_hbm.at[src_v.at[b]], rows_v.at[b], sem).wait()

        @pl.when(j + 2 < cnt)
        def _():
            pltpu.make_async_copy(src_hbm.at[base + j + 2],
                                  src_v.at[b2], sem_idx).wait()
            pltpu.make_async_copy(dst_hbm.at[base + j + 2],
                                  dst_v.at[b2], sem_idx).wait()
            pltpu.make_async_copy(g_hbm.at[src_v.at[b2]],
                                  rows_v.at[b2], sem).start()

        pltpu.sync_copy(rows_v.at[b], acc.at[dst_v.at[b]], add=True)

        @pl.when(j + 3 < cnt)
        def _():
            pltpu.make_async_copy(src_hbm.at[base + j + 3],
                                  src_v.at[b], sem_idx).start()
            pltpu.make_async_copy(dst_hbm.at[base + j + 3],
                                  dst_v.at[b], sem_idx).start()
        return 0
    lax.fori_loop(0, cnt, step, 0)
    plsc.subcore_barrier()

    pltpu.sync_copy(acc.at[pl.ds(s * ROWS_PER_TILE, ROWS_PER_TILE)],
                    out_hbm.at[c, pl.ds(s * ROWS_PER_TILE, ROWS_PER_TILE)])


_BLK = 632


def _scale0(deg_part, x):
    def body(dp, xr, nrm_ref, g1_ref):
        a = dp[...]
        d = a[0, :, 0:1] + a[1, :, 0:1]
        nrm = lax.rsqrt(jnp.maximum(d, 1.0))
        nrm_ref[...] = nrm
        g1_ref[...] = xr[...] * nrm

    return pl.pallas_call(
        body,
        grid=(NPAD // _BLK,),
        in_specs=[pl.BlockSpec((2, _BLK, D), lambda i: (0, i, 0)),
                  pl.BlockSpec((_BLK, D), lambda i: (i, 0))],
        out_specs=(pl.BlockSpec((_BLK, 1), lambda i: (i, 0)),
                   pl.BlockSpec((_BLK, D), lambda i: (i, 0))),
        out_shape=(jax.ShapeDtypeStruct((NPAD, 1), jnp.float32),
                   jax.ShapeDtypeStruct((NPAD, D), jnp.float32)),
    )(deg_part, x)


def _combine1(p, norm):
    def body(pr, nr, h1_ref, g2_ref):
        a = pr[...]
        nrm = nr[...]
        h1 = (a[0] + a[1]) * nrm
        h1_ref[...] = h1
        g2_ref[...] = h1 * nrm

    return pl.pallas_call(
        body,
        grid=(NPAD // _BLK,),
        in_specs=[pl.BlockSpec((2, _BLK, D), lambda i: (0, i, 0)),
                  pl.BlockSpec((_BLK, 1), lambda i: (i, 0))],
        out_specs=(pl.BlockSpec((_BLK, D), lambda i: (i, 0)),
                   pl.BlockSpec((_BLK, D), lambda i: (i, 0))),
        out_shape=(jax.ShapeDtypeStruct((NPAD, D), jnp.float32),
                   jax.ShapeDtypeStruct((NPAD, D), jnp.float32)),
    )(p, norm)


def _final(q, norm, x, h1):
    def body(qr, nr, xr, h1r, o_ref):
        a = qr[...]
        h2 = (a[0] + a[1]) * nr[...]
        o_ref[...] = (xr[...] + h1r[...] + h2) * (1.0 / 3.0)

    return pl.pallas_call(
        body,
        grid=(NPAD // _BLK,),
        in_specs=[pl.BlockSpec((2, _BLK, D), lambda i: (0, i, 0)),
                  pl.BlockSpec((_BLK, 1), lambda i: (i, 0)),
                  pl.BlockSpec((_BLK, D), lambda i: (i, 0)),
                  pl.BlockSpec((_BLK, D), lambda i: (i, 0))],
        out_specs=pl.BlockSpec((_BLK, D), lambda i: (i, 0)),
        out_shape=jax.ShapeDtypeStruct((NPAD, D), jnp.float32),
    )(q, norm, x, h1)


def kernel(features, edge_index):
    ei = edge_index.astype(jnp.int32)
    trash = jnp.full((EPAD - N_EDGES,), NPAD - 1, jnp.int32)
    src = jnp.concatenate([ei[0], trash]).reshape(NCHUNKS_TOTAL, CH)
    dst = jnp.concatenate([ei[1], trash]).reshape(NCHUNKS_TOTAL, CH)
    x = jnp.pad(features, ((0, NPAD - N_NODES), (0, 0)))

    onesz = jnp.stack([jnp.ones((CH, D), jnp.float32),
                       jnp.zeros((CH, D), jnp.float32)])
    deg_part = _deg_kernel(onesz, dst)
    norm, g1 = _scale0(deg_part, x)
    p = _scatter_kernel(g1, src, dst)
    h1, g2 = _combine1(p, norm)
    q = _scatter_kernel(g2, src, dst)
    out = _final(q, norm, x, h1)
    return out[:N_NODES]


# speedup vs baseline: 1.2702x; 1.2702x over previous
import functools

import jax
import jax.numpy as jnp
from jax import lax
from jax.experimental import pallas as pl
from jax.experimental.pallas import tpu as pltpu
from jax.experimental.pallas import tpu_sc as plsc

N_NODES = 10000
D = 128
N_EDGES = 320000

NC = 2
NS = 16
NW = NC * NS

NPAD = 10112
ROWS_PER_TILE = NPAD // NS

CH = 128
NCHUNK = 79
EW = CH * NCHUNK
EPAD = EW * NW
NCHUNKS_TOTAL = EPAD // CH
CPP = 2 * NCHUNK
CN0 = CPP // 2
CN1 = CPP - CN0
_ZSIZES = [CH] * (ROWS_PER_TILE // CH) + (
    [ROWS_PER_TILE % CH] if ROWS_PER_TILE % CH else [])

_MESH = plsc.VectorSubcoreMesh(core_axis_name="c", subcore_axis_name="s")


@functools.partial(
    pl.kernel,
    out_type=jax.ShapeDtypeStruct((NC, NPAD, D), jnp.float32),
    mesh=_MESH,
    scratch_types=[
        pltpu.VMEM((2, CH), jnp.int32),
        pltpu.VMEM((CH, D), jnp.float32),
        pltpu.VMEM_SHARED((NPAD, D), jnp.float32),
        pltpu.SemaphoreType.DMA,
    ],
)
def _deg_kernel(onesz_hbm, dst_hbm, out_hbm, dst_v, ones_v, acc, sem_idx):
    c = lax.axis_index("c")
    s = lax.axis_index("s")
    base = (s * NC + c) * NCHUNK

    pltpu.sync_copy(onesz_hbm.at[1], ones_v)
    off = 0
    for sz in _ZSIZES:
        pltpu.sync_copy(ones_v.at[pl.ds(0, sz)],
                        acc.at[pl.ds(s * ROWS_PER_TILE + off, sz)])
        off += sz
    pltpu.sync_copy(onesz_hbm.at[0], ones_v)
    plsc.subcore_barrier()

    pltpu.sync_copy(dst_hbm.at[base], dst_v.at[0])
    pltpu.make_async_copy(dst_hbm.at[base + 1], dst_v.at[1], sem_idx).start()

    def step(j, _):
        b = lax.rem(j, 2)
        pltpu.sync_copy(ones_v, acc.at[dst_v.at[b]], add=True)

        @pl.when(j + 1 < NCHUNK)
        def _():
            pltpu.make_async_copy(dst_hbm.at[base + j + 1],
                                  dst_v.at[1 - b], sem_idx).wait()

        @pl.when(j + 2 < NCHUNK)
        def _():
            pltpu.make_async_copy(dst_hbm.at[base + j + 2],
                                  dst_v.at[b], sem_idx).start()
        return 0
    lax.fori_loop(0, NCHUNK, step, 0)
    plsc.subcore_barrier()

    pltpu.sync_copy(acc.at[pl.ds(s * ROWS_PER_TILE, ROWS_PER_TILE)],
                    out_hbm.at[c, pl.ds(s * ROWS_PER_TILE, ROWS_PER_TILE)])


@functools.partial(
    pl.kernel,
    out_type=jax.ShapeDtypeStruct((NC, NPAD, D), jnp.float32),
    mesh=_MESH,
    scratch_types=[
        pltpu.VMEM((3, CH), jnp.int32),
        pltpu.VMEM((3, CH), jnp.int32),
        pltpu.VMEM((3, CH, D), jnp.float32),
        pltpu.VMEM_SHARED((NPAD, D), jnp.float32),
        pltpu.SemaphoreType.DMA,
        pltpu.SemaphoreType.DMA,
    ],
)
def _scatter_kernel(g_hbm, src_hbm, dst_hbm, out_hbm,
                    src_v, dst_v, rows_v, acc, sem, sem_idx):
    c = lax.axis_index("c")
    s = lax.axis_index("s")
    base = s * CPP + c * CN0
    cnt = jnp.where(c == 0, CN0, CN1)

    zero16 = jnp.zeros((16,), jnp.float32)

    def fill(r, _):
        def fcol(k, _):
            rows_v[0, r, pl.ds(k * 16, 16)] = zero16
            return 0
        return lax.fori_loop(0, D // 16, fcol, 0)
    lax.fori_loop(0, CH, fill, 0)

    off = 0
    for sz in _ZSIZES:
        pltpu.sync_copy(rows_v.at[0, pl.ds(0, sz)],
                        acc.at[pl.ds(s * ROWS_PER_TILE + off, sz)])
        off += sz
    plsc.subcore_barrier()

    @pl.when(0 < cnt)
    def _():
        pltpu.sync_copy(src_hbm.at[base], src_v.at[0])
        pltpu.sync_copy(dst_hbm.at[base], dst_v.at[0])
        pltpu.make_async_copy(g_hbm.at[src_v.at[0]], rows_v.at[0],
                              sem).start()

    @pl.when(1 < cnt)
    def _():
        pltpu.sync_copy(src_hbm.at[base + 1], src_v.at[1])
        pltpu.sync_copy(dst_hbm.at[base + 1], dst_v.at[1])
        pltpu.make_async_copy(g_hbm.at[src_v.at[1]], rows_v.at[1],
                              sem).start()

    @pl.when(2 < cnt)
    def _():
        pltpu.make_async_copy(src_hbm.at[base + 2], src_v.at[2],
                              sem_idx).start()
        pltpu.make_async_copy(dst_hbm.at[base + 2], dst_v.at[2],
                              sem_idx).start()

    def step(j, _):
        b = lax.rem(j, 3)
        b2 = lax.rem(j + 2, 3)
        pltpu.make_async_copy(g_hbm.at[src_v.at[b]], rows_v.at[b], sem).wait()

        @pl.when(j + 2 < cnt)
        def _():
            pltpu.make_async_copy(src_hbm.at[base + j + 2],
                                  src_v.at[b2], sem_idx).wait()
            pltpu.make_async_copy(dst_hbm.at[base + j + 2],
                                  dst_v.at[b2], sem_idx).wait()
            pltpu.make_async_copy(g_hbm.at[src_v.at[b2]],
                                  rows_v.at[b2], sem).start()

        pltpu.sync_copy(rows_v.at[b], acc.at[dst_v.at[b]], add=True)

        @pl.when(j + 3 < cnt)
        def _():
            pltpu.make_async_copy(src_hbm.at[base + j + 3],
                                  src_v.at[b], sem_idx).start()
            pltpu.make_async_copy(dst_hbm.at[base + j + 3],
                                  dst_v.at[b], sem_idx).start()
        return 0
    lax.fori_loop(0, cnt, step, 0)
    plsc.subcore_barrier()

    pltpu.sync_copy(acc.at[pl.ds(s * ROWS_PER_TILE, ROWS_PER_TILE)],
                    out_hbm.at[c, pl.ds(s * ROWS_PER_TILE, ROWS_PER_TILE)])


_BLK = 632


def _scale0(deg_part, x):
    def body(dp, xr, nrm_ref, g1_ref):
        a = dp[...]
        d = a[0, :, 0:1] + a[1, :, 0:1]
        nrm = lax.rsqrt(jnp.maximum(d, 1.0))
        nrm_ref[...] = nrm
        g1_ref[...] = xr[...] * nrm

    return pl.pallas_call(
        body,
        grid=(NPAD // _BLK,),
        in_specs=[pl.BlockSpec((2, _BLK, D), lambda i: (0, i, 0)),
                  pl.BlockSpec((_BLK, D), lambda i: (i, 0))],
        out_specs=(pl.BlockSpec((_BLK, 1), lambda i: (i, 0)),
                   pl.BlockSpec((_BLK, D), lambda i: (i, 0))),
        out_shape=(jax.ShapeDtypeStruct((NPAD, 1), jnp.float32),
                   jax.ShapeDtypeStruct((NPAD, D), jnp.float32)),
    )(deg_part, x)


def _combine1(p, norm):
    def body(pr, nr, h1_ref, g2_ref):
        a = pr[...]
        nrm = nr[...]
        h1 = (a[0] + a[1]) * nrm
        h1_ref[...] = h1
        g2_ref[...] = h1 * nrm

    return pl.pallas_call(
        body,
        grid=(NPAD // _BLK,),
        in_specs=[pl.BlockSpec((2, _BLK, D), lambda i: (0, i, 0)),
                  pl.BlockSpec((_BLK, 1), lambda i: (i, 0))],
        out_specs=(pl.BlockSpec((_BLK, D), lambda i: (i, 0)),
                   pl.BlockSpec((_BLK, D), lambda i: (i, 0))),
        out_shape=(jax.ShapeDtypeStruct((NPAD, D), jnp.float32),
                   jax.ShapeDtypeStruct((NPAD, D), jnp.float32)),
    )(p, norm)


def _final(q, norm, x, h1):
    def body(qr, nr, xr, h1r, o_ref):
        a = qr[...]
        h2 = (a[0] + a[1]) * nr[...]
        o_ref[...] = (xr[...] + h1r[...] + h2) * (1.0 / 3.0)

    return pl.pallas_call(
        body,
        grid=(NPAD // _BLK,),
        in_specs=[pl.BlockSpec((2, _BLK, D), lambda i: (0, i, 0)),
                  pl.BlockSpec((_BLK, 1), lambda i: (i, 0)),
                  pl.BlockSpec((_BLK, D), lambda i: (i, 0)),
                  pl.BlockSpec((_BLK, D), lambda i: (i, 0))],
        out_specs=pl.BlockSpec((_BLK, D), lambda i: (i, 0)),
        out_shape=jax.ShapeDtypeStruct((NPAD, D), jnp.float32),
    )(q, norm, x, h1)


def kernel(features, edge_index):
    ei = edge_index.astype(jnp.int32)
    trash = jnp.full((EPAD - N_EDGES,), NPAD - 1, jnp.int32)
    src = jnp.concatenate([ei[0], trash]).reshape(NCHUNKS_TOTAL, CH)
    dst = jnp.concatenate([ei[1], trash]).reshape(NCHUNKS_TOTAL, CH)
    x = jnp.pad(features, ((0, NPAD - N_NODES), (0, 0)))

    onesz = jnp.stack([jnp.ones((CH, D), jnp.float32),
                       jnp.zeros((CH, D), jnp.float32)])
    deg_part = _deg_kernel(onesz, dst)
    norm, g1 = _scale0(deg_part, x)
    p = _scatter_kernel(g1, src, dst)
    h1, g2 = _combine1(p, norm)
    q = _scatter_kernel(g2, src, dst)
    out = _final(q, norm, x, h1)
    return out[:N_NODES]


# speedup vs baseline: 1.3103x; 1.0315x over previous
import functools

import jax
import jax.numpy as jnp
from jax import lax
from jax.experimental import pallas as pl
from jax.experimental.pallas import tpu as pltpu
from jax.experimental.pallas import tpu_sc as plsc

N_NODES = 10000
D = 128
N_EDGES = 320000

NC = 2
NS = 16
NW = NC * NS

NPAD = 10112
ROWS_PER_TILE = NPAD // NS

CH = 128
NCHUNK = 79
EW = CH * NCHUNK
EPAD = EW * NW
NCHUNKS_TOTAL = EPAD // CH
CPP = 2 * NCHUNK
CN0 = 88
CN1 = CPP - CN0
_ZSIZES = [CH] * (ROWS_PER_TILE // CH) + (
    [ROWS_PER_TILE % CH] if ROWS_PER_TILE % CH else [])

_MESH = plsc.VectorSubcoreMesh(core_axis_name="c", subcore_axis_name="s")


@functools.partial(
    pl.kernel,
    out_type=jax.ShapeDtypeStruct((NC, NPAD, D), jnp.float32),
    mesh=_MESH,
    scratch_types=[
        pltpu.VMEM((2, CH), jnp.int32),
        pltpu.VMEM((CH, D), jnp.float32),
        pltpu.VMEM_SHARED((NPAD, D), jnp.float32),
        pltpu.SemaphoreType.DMA,
    ],
)
def _deg_kernel(onesz_hbm, dst_hbm, out_hbm, dst_v, ones_v, acc, sem_idx):
    c = lax.axis_index("c")
    s = lax.axis_index("s")
    base = (s * NC + c) * NCHUNK

    pltpu.sync_copy(onesz_hbm.at[1], ones_v)
    off = 0
    for sz in _ZSIZES:
        pltpu.sync_copy(ones_v.at[pl.ds(0, sz)],
                        acc.at[pl.ds(s * ROWS_PER_TILE + off, sz)])
        off += sz
    pltpu.sync_copy(onesz_hbm.at[0], ones_v)
    plsc.subcore_barrier()

    pltpu.sync_copy(dst_hbm.at[base], dst_v.at[0])
    pltpu.make_async_copy(dst_hbm.at[base + 1], dst_v.at[1], sem_idx).start()

    def step(j, _):
        b = lax.rem(j, 2)
        pltpu.sync_copy(ones_v, acc.at[dst_v.at[b]], add=True)

        @pl.when(j + 1 < NCHUNK)
        def _():
            pltpu.make_async_copy(dst_hbm.at[base + j + 1],
                                  dst_v.at[1 - b], sem_idx).wait()

        @pl.when(j + 2 < NCHUNK)
        def _():
            pltpu.make_async_copy(dst_hbm.at[base + j + 2],
                                  dst_v.at[b], sem_idx).start()
        return 0
    lax.fori_loop(0, NCHUNK, step, 0)
    plsc.subcore_barrier()

    pltpu.sync_copy(acc.at[pl.ds(s * ROWS_PER_TILE, ROWS_PER_TILE)],
                    out_hbm.at[c, pl.ds(s * ROWS_PER_TILE, ROWS_PER_TILE)])


@functools.partial(
    pl.kernel,
    out_type=jax.ShapeDtypeStruct((NC, NPAD, D), jnp.float32),
    mesh=_MESH,
    scratch_types=[
        pltpu.VMEM((3, CH), jnp.int32),
        pltpu.VMEM((3, CH), jnp.int32),
        pltpu.VMEM((3, CH, D), jnp.float32),
        pltpu.VMEM_SHARED((NPAD, D), jnp.float32),
        pltpu.SemaphoreType.DMA,
        pltpu.SemaphoreType.DMA,
    ],
)
def _scatter_kernel(g_hbm, src_hbm, dst_hbm, out_hbm,
                    src_v, dst_v, rows_v, acc, sem, sem_idx):
    c = lax.axis_index("c")
    s = lax.axis_index("s")
    base = s * CPP + c * CN0
    cnt = jnp.where(c == 0, CN0, CN1)

    zero16 = jnp.zeros((16,), jnp.float32)

    def fill(r, _):
        def fcol(k, _):
            rows_v[0, r, pl.ds(k * 16, 16)] = zero16
            return 0
        return lax.fori_loop(0, D // 16, fcol, 0)
    lax.fori_loop(0, CH, fill, 0)

    off = 0
    for sz in _ZSIZES:
        pltpu.sync_copy(rows_v.at[0, pl.ds(0, sz)],
                        acc.at[pl.ds(s * ROWS_PER_TILE + off, sz)])
        off += sz
    plsc.subcore_barrier()

    @pl.when(0 < cnt)
    def _():
        pltpu.sync_copy(src_hbm.at[base], src_v.at[0])
        pltpu.sync_copy(dst_hbm.at[base], dst_v.at[0])
        pltpu.make_async_copy(g_hbm.at[src_v.at[0]], rows_v.at[0],
                              sem).start()

    @pl.when(1 < cnt)
    def _():
        pltpu.sync_copy(src_hbm.at[base + 1], src_v.at[1])
        pltpu.sync_copy(dst_hbm.at[base + 1], dst_v.at[1])
        pltpu.make_async_copy(g_hbm.at[src_v.at[1]], rows_v.at[1],
                              sem).start()

    @pl.when(2 < cnt)
    def _():
        pltpu.make_async_copy(src_hbm.at[base + 2], src_v.at[2],
                              sem_idx).start()
        pltpu.make_async_copy(dst_hbm.at[base + 2], dst_v.at[2],
                              sem_idx).start()

    def step(j, _):
        b = lax.rem(j, 3)
        b2 = lax.rem(j + 2, 3)
        pltpu.make_async_copy(g_hbm.at[src_v.at[b]], rows_v.at[b], sem).wait()

        @pl.when(j + 2 < cnt)
        def _():
            pltpu.make_async_copy(src_hbm.at[base + j + 2],
                                  src_v.at[b2], sem_idx).wait()
            pltpu.make_async_copy(dst_hbm.at[base + j + 2],
                                  dst_v.at[b2], sem_idx).wait()
            pltpu.make_async_copy(g_hbm.at[src_v.at[b2]],
                                  rows_v.at[b2], sem).start()

        pltpu.sync_copy(rows_v.at[b], acc.at[dst_v.at[b]], add=True)

        @pl.when(j + 3 < cnt)
        def _():
            pltpu.make_async_copy(src_hbm.at[base + j + 3],
                                  src_v.at[b], sem_idx).start()
            pltpu.make_async_copy(dst_hbm.at[base + j + 3],
                                  dst_v.at[b], sem_idx).start()
        return 0
    lax.fori_loop(0, cnt, step, 0)
    plsc.subcore_barrier()

    pltpu.sync_copy(acc.at[pl.ds(s * ROWS_PER_TILE, ROWS_PER_TILE)],
                    out_hbm.at[c, pl.ds(s * ROWS_PER_TILE, ROWS_PER_TILE)])


_BLK = 632


def _scale0(deg_part, x):
    def body(dp, xr, nrm_ref, g1_ref):
        a = dp[...]
        d = a[0, :, 0:1] + a[1, :, 0:1]
        nrm = lax.rsqrt(jnp.maximum(d, 1.0))
        nrm_ref[...] = nrm
        g1_ref[...] = xr[...] * nrm

    return pl.pallas_call(
        body,
        grid=(NPAD // _BLK,),
        in_specs=[pl.BlockSpec((2, _BLK, D), lambda i: (0, i, 0)),
                  pl.BlockSpec((_BLK, D), lambda i: (i, 0))],
        out_specs=(pl.BlockSpec((_BLK, 1), lambda i: (i, 0)),
                   pl.BlockSpec((_BLK, D), lambda i: (i, 0))),
        out_shape=(jax.ShapeDtypeStruct((NPAD, 1), jnp.float32),
                   jax.ShapeDtypeStruct((NPAD, D), jnp.float32)),
    )(deg_part, x)


def _combine1(p, norm):
    def body(pr, nr, h1_ref, g2_ref):
        a = pr[...]
        nrm = nr[...]
        h1 = (a[0] + a[1]) * nrm
        h1_ref[...] = h1
        g2_ref[...] = h1 * nrm

    return pl.pallas_call(
        body,
        grid=(NPAD // _BLK,),
        in_specs=[pl.BlockSpec((2, _BLK, D), lambda i: (0, i, 0)),
                  pl.BlockSpec((_BLK, 1), lambda i: (i, 0))],
        out_specs=(pl.BlockSpec((_BLK, D), lambda i: (i, 0)),
                   pl.BlockSpec((_BLK, D), lambda i: (i, 0))),
        out_shape=(jax.ShapeDtypeStruct((NPAD, D), jnp.float32),
                   jax.ShapeDtypeStruct((NPAD, D), jnp.float32)),
    )(p, norm)


def _final(q, norm, x, h1):
    def body(qr, nr, xr, h1r, o_ref):
        a = qr[...]
        h2 = (a[0] + a[1]) * nr[...]
        o_ref[...] = (xr[...] + h1r[...] + h2) * (1.0 / 3.0)

    return pl.pallas_call(
        body,
        grid=(NPAD // _BLK,),
        in_specs=[pl.BlockSpec((2, _BLK, D), lambda i: (0, i, 0)),
                  pl.BlockSpec((_BLK, 1), lambda i: (i, 0)),
                  pl.BlockSpec((_BLK, D), lambda i: (i, 0)),
                  pl.BlockSpec((_BLK, D), lambda i: (i, 0))],
        out_specs=pl.BlockSpec((_BLK, D), lambda i: (i, 0)),
        out_shape=jax.ShapeDtypeStruct((NPAD, D), jnp.float32),
    )(q, norm, x, h1)


def kernel(features, edge_index):
    ei = edge_index.astype(jnp.int32)
    trash = jnp.full((EPAD - N_EDGES,), NPAD - 1, jnp.int32)
    src = jnp.concatenate([ei[0], trash]).reshape(NCHUNKS_TOTAL, CH)
    dst = jnp.concatenate([ei[1], trash]).reshape(NCHUNKS_TOTAL, CH)
    x = jnp.pad(features, ((0, NPAD - N_NODES), (0, 0)))

    onesz = jnp.stack([jnp.ones((CH, D), jnp.float32),
                       jnp.zeros((CH, D), jnp.float32)])
    deg_part = _deg_kernel(onesz, dst)
    norm, g1 = _scale0(deg_part, x)
    p = _scatter_kernel(g1, src, dst)
    h1, g2 = _combine1(p, norm)
    q = _scatter_kernel(g2, src, dst)
    out = _final(q, norm, x, h1)
    return out[:N_NODES]


# speedup vs baseline: 1.3458x; 1.0271x over previous
import functools

import jax
import jax.numpy as jnp
from jax import lax
from jax.experimental import pallas as pl
from jax.experimental.pallas import tpu as pltpu
from jax.experimental.pallas import tpu_sc as plsc

N_NODES = 10000
D = 128
N_EDGES = 320000

NC = 2
NS = 16
NW = NC * NS

NPAD = 10112
ROWS_PER_TILE = NPAD // NS

CH = 128
NCHUNK = 79
EW = CH * NCHUNK
EPAD = EW * NW
NCHUNKS_TOTAL = EPAD // CH
CPP = 2 * NCHUNK
CN0 = 96
CN1 = CPP - CN0
_ZSIZES = [CH] * (ROWS_PER_TILE // CH) + (
    [ROWS_PER_TILE % CH] if ROWS_PER_TILE % CH else [])

_MESH = plsc.VectorSubcoreMesh(core_axis_name="c", subcore_axis_name="s")


@functools.partial(
    pl.kernel,
    out_type=jax.ShapeDtypeStruct((NC, NPAD, D), jnp.float32),
    mesh=_MESH,
    scratch_types=[
        pltpu.VMEM((2, CH), jnp.int32),
        pltpu.VMEM((CH, D), jnp.float32),
        pltpu.VMEM_SHARED((NPAD, D), jnp.float32),
        pltpu.SemaphoreType.DMA,
    ],
)
def _deg_kernel(onesz_hbm, dst_hbm, out_hbm, dst_v, ones_v, acc, sem_idx):
    c = lax.axis_index("c")
    s = lax.axis_index("s")
    base = (s * NC + c) * NCHUNK

    pltpu.sync_copy(onesz_hbm.at[1], ones_v)
    off = 0
    for sz in _ZSIZES:
        pltpu.sync_copy(ones_v.at[pl.ds(0, sz)],
                        acc.at[pl.ds(s * ROWS_PER_TILE + off, sz)])
        off += sz
    pltpu.sync_copy(onesz_hbm.at[0], ones_v)
    plsc.subcore_barrier()

    pltpu.sync_copy(dst_hbm.at[base], dst_v.at[0])
    pltpu.make_async_copy(dst_hbm.at[base + 1], dst_v.at[1], sem_idx).start()

    def step(j, _):
        b = lax.rem(j, 2)
        pltpu.sync_copy(ones_v, acc.at[dst_v.at[b]], add=True)

        @pl.when(j + 1 < NCHUNK)
        def _():
            pltpu.make_async_copy(dst_hbm.at[base + j + 1],
                                  dst_v.at[1 - b], sem_idx).wait()

        @pl.when(j + 2 < NCHUNK)
        def _():
            pltpu.make_async_copy(dst_hbm.at[base + j + 2],
                                  dst_v.at[b], sem_idx).start()
        return 0
    lax.fori_loop(0, NCHUNK, step, 0)
    plsc.subcore_barrier()

    pltpu.sync_copy(acc.at[pl.ds(s * ROWS_PER_TILE, ROWS_PER_TILE)],
                    out_hbm.at[c, pl.ds(s * ROWS_PER_TILE, ROWS_PER_TILE)])


@functools.partial(
    pl.kernel,
    out_type=jax.ShapeDtypeStruct((NC, NPAD, D), jnp.float32),
    mesh=_MESH,
    scratch_types=[
        pltpu.VMEM((3, CH), jnp.int32),
        pltpu.VMEM((3, CH), jnp.int32),
        pltpu.VMEM((3, CH, D), jnp.float32),
        pltpu.VMEM_SHARED((NPAD, D), jnp.float32),
        pltpu.SemaphoreType.DMA,
        pltpu.SemaphoreType.DMA,
    ],
)
def _scatter_kernel(g_hbm, src_hbm, dst_hbm, out_hbm,
                    src_v, dst_v, rows_v, acc, sem, sem_idx):
    c = lax.axis_index("c")
    s = lax.axis_index("s")
    base = s * CPP + c * CN0
    cnt = jnp.where(c == 0, CN0, CN1)

    zero16 = jnp.zeros((16,), jnp.float32)

    def fill(r, _):
        def fcol(k, _):
            rows_v[0, r, pl.ds(k * 16, 16)] = zero16
            return 0
        return lax.fori_loop(0, D // 16, fcol, 0)
    lax.fori_loop(0, CH, fill, 0)

    off = 0
    for sz in _ZSIZES:
        pltpu.sync_copy(rows_v.at[0, pl.ds(0, sz)],
                        acc.at[pl.ds(s * ROWS_PER_TILE + off, sz)])
        off += sz
    plsc.subcore_barrier()

    @pl.when(0 < cnt)
    def _():
        pltpu.sync_copy(src_hbm.at[base], src_v.at[0])
        pltpu.sync_copy(dst_hbm.at[base], dst_v.at[0])
        pltpu.make_async_copy(g_hbm.at[src_v.at[0]], rows_v.at[0],
                              sem).start()

    @pl.when(1 < cnt)
    def _():
        pltpu.sync_copy(src_hbm.at[base + 1], src_v.at[1])
        pltpu.sync_copy(dst_hbm.at[base + 1], dst_v.at[1])
        pltpu.make_async_copy(g_hbm.at[src_v.at[1]], rows_v.at[1],
                              sem).start()

    @pl.when(2 < cnt)
    def _():
        pltpu.make_async_copy(src_hbm.at[base + 2], src_v.at[2],
                              sem_idx).start()
        pltpu.make_async_copy(dst_hbm.at[base + 2], dst_v.at[2],
                              sem_idx).start()

    def step(j, _):
        b = lax.rem(j, 3)
        b2 = lax.rem(j + 2, 3)
        pltpu.make_async_copy(g_hbm.at[src_v.at[b]], rows_v.at[b], sem).wait()

        @pl.when(j + 2 < cnt)
        def _():
            pltpu.make_async_copy(src_hbm.at[base + j + 2],
                                  src_v.at[b2], sem_idx).wait()
            pltpu.make_async_copy(dst_hbm.at[base + j + 2],
                                  dst_v.at[b2], sem_idx).wait()
            pltpu.make_async_copy(g_hbm.at[src_v.at[b2]],
                                  rows_v.at[b2], sem).start()

        pltpu.sync_copy(rows_v.at[b], acc.at[dst_v.at[b]], add=True)

        @pl.when(j + 3 < cnt)
        def _():
            pltpu.make_async_copy(src_hbm.at[base + j + 3],
                                  src_v.at[b], sem_idx).start()
            pltpu.make_async_copy(dst_hbm.at[base + j + 3],
                                  dst_v.at[b], sem_idx).start()
        return 0
    lax.fori_loop(0, cnt, step, 0)
    plsc.subcore_barrier()

    pltpu.sync_copy(acc.at[pl.ds(s * ROWS_PER_TILE, ROWS_PER_TILE)],
                    out_hbm.at[c, pl.ds(s * ROWS_PER_TILE, ROWS_PER_TILE)])


_BLK = 632


def _scale0(deg_part, x):
    def body(dp, xr, nrm_ref, g1_ref):
        a = dp[...]
        d = a[0, :, 0:1] + a[1, :, 0:1]
        nrm = lax.rsqrt(jnp.maximum(d, 1.0))
        nrm_ref[...] = nrm
        g1_ref[...] = xr[...] * nrm

    return pl.pallas_call(
        body,
        grid=(NPAD // _BLK,),
        in_specs=[pl.BlockSpec((2, _BLK, D), lambda i: (0, i, 0)),
                  pl.BlockSpec((_BLK, D), lambda i: (i, 0))],
        out_specs=(pl.BlockSpec((_BLK, 1), lambda i: (i, 0)),
                   pl.BlockSpec((_BLK, D), lambda i: (i, 0))),
        out_shape=(jax.ShapeDtypeStruct((NPAD, 1), jnp.float32),
                   jax.ShapeDtypeStruct((NPAD, D), jnp.float32)),
    )(deg_part, x)


def _combine1(p, norm):
    def body(pr, nr, h1_ref, g2_ref):
        a = pr[...]
        nrm = nr[...]
        h1 = (a[0] + a[1]) * nrm
        h1_ref[...] = h1
        g2_ref[...] = h1 * nrm

    return pl.pallas_call(
        body,
        grid=(NPAD // _BLK,),
        in_specs=[pl.BlockSpec((2, _BLK, D), lambda i: (0, i, 0)),
                  pl.BlockSpec((_BLK, 1), lambda i: (i, 0))],
        out_specs=(pl.BlockSpec((_BLK, D), lambda i: (i, 0)),
                   pl.BlockSpec((_BLK, D), lambda i: (i, 0))),
        out_shape=(jax.ShapeDtypeStruct((NPAD, D), jnp.float32),
                   jax.ShapeDtypeStruct((NPAD, D), jnp.float32)),
    )(p, norm)


def _final(q, norm, x, h1):
    def body(qr, nr, xr, h1r, o_ref):
        a = qr[...]
        h2 = (a[0] + a[1]) * nr[...]
        o_ref[...] = (xr[...] + h1r[...] + h2) * (1.0 / 3.0)

    return pl.pallas_call(
        body,
        grid=(NPAD // _BLK,),
        in_specs=[pl.BlockSpec((2, _BLK, D), lambda i: (0, i, 0)),
                  pl.BlockSpec((_BLK, 1), lambda i: (i, 0)),
                  pl.BlockSpec((_BLK, D), lambda i: (i, 0)),
                  pl.BlockSpec((_BLK, D), lambda i: (i, 0))],
        out_specs=pl.BlockSpec((_BLK, D), lambda i: (i, 0)),
        out_shape=jax.ShapeDtypeStruct((NPAD, D), jnp.float32),
    )(q, norm, x, h1)


def kernel(features, edge_index):
    ei = edge_index.astype(jnp.int32)
    trash = jnp.full((EPAD - N_EDGES,), NPAD - 1, jnp.int32)
    src = jnp.concatenate([ei[0], trash]).reshape(NCHUNKS_TOTAL, CH)
    dst = jnp.concatenate([ei[1], trash]).reshape(NCHUNKS_TOTAL, CH)
    x = jnp.pad(features, ((0, NPAD - N_NODES), (0, 0)))

    onesz = jnp.stack([jnp.ones((CH, D), jnp.float32),
                       jnp.zeros((CH, D), jnp.float32)])
    deg_part = _deg_kernel(onesz, dst)
    norm, g1 = _scale0(deg_part, x)
    p = _scatter_kernel(g1, src, dst)
    h1, g2 = _combine1(p, norm)
    q = _scatter_kernel(g2, src, dst)
    out = _final(q, norm, x, h1)
    return out[:N_NODES]


# speedup vs baseline: 1.3818x; 1.0268x over previous
import functools

import jax
import jax.numpy as jnp
from jax import lax
from jax.experimental import pallas as pl
from jax.experimental.pallas import tpu as pltpu
from jax.experimental.pallas import tpu_sc as plsc

N_NODES = 10000
D = 128
N_EDGES = 320000

NC = 2
NS = 16
NW = NC * NS

NPAD = 10112
ROWS_PER_TILE = NPAD // NS

CH = 128
NCHUNK = 79
EW = CH * NCHUNK
EPAD = EW * NW
NCHUNKS_TOTAL = EPAD // CH
CPP = 2 * NCHUNK
CN0 = 104
CN1 = CPP - CN0
_ZSIZES = [CH] * (ROWS_PER_TILE // CH) + (
    [ROWS_PER_TILE % CH] if ROWS_PER_TILE % CH else [])

_MESH = plsc.VectorSubcoreMesh(core_axis_name="c", subcore_axis_name="s")


@functools.partial(
    pl.kernel,
    out_type=jax.ShapeDtypeStruct((NC, NPAD, D), jnp.float32),
    mesh=_MESH,
    scratch_types=[
        pltpu.VMEM((2, CH), jnp.int32),
        pltpu.VMEM((CH, D), jnp.float32),
        pltpu.VMEM_SHARED((NPAD, D), jnp.float32),
        pltpu.SemaphoreType.DMA,
    ],
)
def _deg_kernel(onesz_hbm, dst_hbm, out_hbm, dst_v, ones_v, acc, sem_idx):
    c = lax.axis_index("c")
    s = lax.axis_index("s")
    base = (s * NC + c) * NCHUNK

    pltpu.sync_copy(onesz_hbm.at[1], ones_v)
    off = 0
    for sz in _ZSIZES:
        pltpu.sync_copy(ones_v.at[pl.ds(0, sz)],
                        acc.at[pl.ds(s * ROWS_PER_TILE + off, sz)])
        off += sz
    pltpu.sync_copy(onesz_hbm.at[0], ones_v)
    plsc.subcore_barrier()

    pltpu.sync_copy(dst_hbm.at[base], dst_v.at[0])
    pltpu.make_async_copy(dst_hbm.at[base + 1], dst_v.at[1], sem_idx).start()

    def step(j, _):
        b = lax.rem(j, 2)
        pltpu.sync_copy(ones_v, acc.at[dst_v.at[b]], add=True)

        @pl.when(j + 1 < NCHUNK)
        def _():
            pltpu.make_async_copy(dst_hbm.at[base + j + 1],
                                  dst_v.at[1 - b], sem_idx).wait()

        @pl.when(j + 2 < NCHUNK)
        def _():
            pltpu.make_async_copy(dst_hbm.at[base + j + 2],
                                  dst_v.at[b], sem_idx).start()
        return 0
    lax.fori_loop(0, NCHUNK, step, 0)
    plsc.subcore_barrier()

    pltpu.sync_copy(acc.at[pl.ds(s * ROWS_PER_TILE, ROWS_PER_TILE)],
                    out_hbm.at[c, pl.ds(s * ROWS_PER_TILE, ROWS_PER_TILE)])


@functools.partial(
    pl.kernel,
    out_type=jax.ShapeDtypeStruct((NC, NPAD, D), jnp.float32),
    mesh=_MESH,
    scratch_types=[
        pltpu.VMEM((3, CH), jnp.int32),
        pltpu.VMEM((3, CH), jnp.int32),
        pltpu.VMEM((3, CH, D), jnp.float32),
        pltpu.VMEM_SHARED((NPAD, D), jnp.float32),
        pltpu.SemaphoreType.DMA,
        pltpu.SemaphoreType.DMA,
    ],
)
def _scatter_kernel(g_hbm, src_hbm, dst_hbm, out_hbm,
                    src_v, dst_v, rows_v, acc, sem, sem_idx):
    c = lax.axis_index("c")
    s = lax.axis_index("s")
    base = s * CPP + c * CN0
    cnt = jnp.where(c == 0, CN0, CN1)

    zero16 = jnp.zeros((16,), jnp.float32)

    def fill(r, _):
        def fcol(k, _):
            rows_v[0, r, pl.ds(k * 16, 16)] = zero16
            return 0
        return lax.fori_loop(0, D // 16, fcol, 0)
    lax.fori_loop(0, CH, fill, 0)

    off = 0
    for sz in _ZSIZES:
        pltpu.sync_copy(rows_v.at[0, pl.ds(0, sz)],
                        acc.at[pl.ds(s * ROWS_PER_TILE + off, sz)])
        off += sz
    plsc.subcore_barrier()

    @pl.when(0 < cnt)
    def _():
        pltpu.sync_copy(src_hbm.at[base], src_v.at[0])
        pltpu.sync_copy(dst_hbm.at[base], dst_v.at[0])
        pltpu.make_async_copy(g_hbm.at[src_v.at[0]], rows_v.at[0],
                              sem).start()

    @pl.when(1 < cnt)
    def _():
        pltpu.sync_copy(src_hbm.at[base + 1], src_v.at[1])
        pltpu.sync_copy(dst_hbm.at[base + 1], dst_v.at[1])
        pltpu.make_async_copy(g_hbm.at[src_v.at[1]], rows_v.at[1],
                              sem).start()

    @pl.when(2 < cnt)
    def _():
        pltpu.make_async_copy(src_hbm.at[base + 2], src_v.at[2],
                              sem_idx).start()
        pltpu.make_async_copy(dst_hbm.at[base + 2], dst_v.at[2],
                              sem_idx).start()

    def step(j, _):
        b = lax.rem(j, 3)
        b2 = lax.rem(j + 2, 3)
        pltpu.make_async_copy(g_hbm.at[src_v.at[b]], rows_v.at[b], sem).wait()

        @pl.when(j + 2 < cnt)
        def _():
            pltpu.make_async_copy(src_hbm.at[base + j + 2],
                                  src_v.at[b2], sem_idx).wait()
            pltpu.make_async_copy(dst_hbm.at[base + j + 2],
                                  dst_v.at[b2], sem_idx).wait()
            pltpu.make_async_copy(g_hbm.at[src_v.at[b2]],
                                  rows_v.at[b2], sem).start()

        pltpu.sync_copy(rows_v.at[b], acc.at[dst_v.at[b]], add=True)

        @pl.when(j + 3 < cnt)
        def _():
            pltpu.make_async_copy(src_hbm.at[base + j + 3],
                                  src_v.at[b], sem_idx).start()
            pltpu.make_async_copy(dst_hbm.at[base + j + 3],
                                  dst_v.at[b], sem_idx).start()
        return 0
    lax.fori_loop(0, cnt, step, 0)
    plsc.subcore_barrier()

    pltpu.sync_copy(acc.at[pl.ds(s * ROWS_PER_TILE, ROWS_PER_TILE)],
                    out_hbm.at[c, pl.ds(s * ROWS_PER_TILE, ROWS_PER_TILE)])


_BLK = 632


def _scale0(deg_part, x):
    def body(dp, xr, nrm_ref, g1_ref):
        a = dp[...]
        d = a[0, :, 0:1] + a[1, :, 0:1]
        nrm = lax.rsqrt(jnp.maximum(d, 1.0))
        nrm_ref[...] = nrm
        g1_ref[...] = xr[...] * nrm

    return pl.pallas_call(
        body,
        grid=(NPAD // _BLK,),
        in_specs=[pl.BlockSpec((2, _BLK, D), lambda i: (0, i, 0)),
                  pl.BlockSpec((_BLK, D), lambda i: (i, 0))],
        out_specs=(pl.BlockSpec((_BLK, 1), lambda i: (i, 0)),
                   pl.BlockSpec((_BLK, D), lambda i: (i, 0))),
        out_shape=(jax.ShapeDtypeStruct((NPAD, 1), jnp.float32),
                   jax.ShapeDtypeStruct((NPAD, D), jnp.float32)),
    )(deg_part, x)


def _combine1(p, norm):
    def body(pr, nr, h1_ref, g2_ref):
        a = pr[...]
        nrm = nr[...]
        h1 = (a[0] + a[1]) * nrm
        h1_ref[...] = h1
        g2_ref[...] = h1 * nrm

    return pl.pallas_call(
        body,
        grid=(NPAD // _BLK,),
        in_specs=[pl.BlockSpec((2, _BLK, D), lambda i: (0, i, 0)),
                  pl.BlockSpec((_BLK, 1), lambda i: (i, 0))],
        out_specs=(pl.BlockSpec((_BLK, D), lambda i: (i, 0)),
                   pl.BlockSpec((_BLK, D), lambda i: (i, 0))),
        out_shape=(jax.ShapeDtypeStruct((NPAD, D), jnp.float32),
                   jax.ShapeDtypeStruct((NPAD, D), jnp.float32)),
    )(p, norm)


def _final(q, norm, x, h1):
    def body(qr, nr, xr, h1r, o_ref):
        a = qr[...]
        h2 = (a[0] + a[1]) * nr[...]
        o_ref[...] = (xr[...] + h1r[...] + h2) * (1.0 / 3.0)

    return pl.pallas_call(
        body,
        grid=(NPAD // _BLK,),
        in_specs=[pl.BlockSpec((2, _BLK, D), lambda i: (0, i, 0)),
                  pl.BlockSpec((_BLK, 1), lambda i: (i, 0)),
                  pl.BlockSpec((_BLK, D), lambda i: (i, 0)),
                  pl.BlockSpec((_BLK, D), lambda i: (i, 0))],
        out_specs=pl.BlockSpec((_BLK, D), lambda i: (i, 0)),
        out_shape=jax.ShapeDtypeStruct((NPAD, D), jnp.float32),
    )(q, norm, x, h1)


def kernel(features, edge_index):
    ei = edge_index.astype(jnp.int32)
    trash = jnp.full((EPAD - N_EDGES,), NPAD - 1, jnp.int32)
    src = jnp.concatenate([ei[0], trash]).reshape(NCHUNKS_TOTAL, CH)
    dst = jnp.concatenate([ei[1], trash]).reshape(NCHUNKS_TOTAL, CH)
    x = jnp.pad(features, ((0, NPAD - N_NODES), (0, 0)))

    onesz = jnp.stack([jnp.ones((CH, D), jnp.float32),
                       jnp.zeros((CH, D), jnp.float32)])
    deg_part = _deg_kernel(onesz, dst)
    norm, g1 = _scale0(deg_part, x)
    p = _scatter_kernel(g1, src, dst)
    h1, g2 = _combine1(p, norm)
    q = _scatter_kernel(g2, src, dst)
    out = _final(q, norm, x, h1)
    return out[:N_NODES]


# speedup vs baseline: 1.4320x; 1.0363x over previous
import functools

import jax
import jax.numpy as jnp
from jax import lax
from jax.experimental import pallas as pl
from jax.experimental.pallas import tpu as pltpu
from jax.experimental.pallas import tpu_sc as plsc

N_NODES = 10000
D = 128
N_EDGES = 320000

NC = 2
NS = 16
NW = NC * NS

NPAD = 10112
ROWS_PER_TILE = NPAD // NS

CH = 128
NCHUNK = 79
EW = CH * NCHUNK
EPAD = EW * NW
NCHUNKS_TOTAL = EPAD // CH
CPP = 2 * NCHUNK
CN0 = 114
CN1 = CPP - CN0
_ZSIZES = [CH] * (ROWS_PER_TILE // CH) + (
    [ROWS_PER_TILE % CH] if ROWS_PER_TILE % CH else [])

_MESH = plsc.VectorSubcoreMesh(core_axis_name="c", subcore_axis_name="s")


@functools.partial(
    pl.kernel,
    out_type=jax.ShapeDtypeStruct((NC, NPAD, D), jnp.float32),
    mesh=_MESH,
    scratch_types=[
        pltpu.VMEM((2, CH), jnp.int32),
        pltpu.VMEM((CH, D), jnp.float32),
        pltpu.VMEM_SHARED((NPAD, D), jnp.float32),
        pltpu.SemaphoreType.DMA,
    ],
)
def _deg_kernel(onesz_hbm, dst_hbm, out_hbm, dst_v, ones_v, acc, sem_idx):
    c = lax.axis_index("c")
    s = lax.axis_index("s")
    base = (s * NC + c) * NCHUNK

    pltpu.sync_copy(onesz_hbm.at[1], ones_v)
    off = 0
    for sz in _ZSIZES:
        pltpu.sync_copy(ones_v.at[pl.ds(0, sz)],
                        acc.at[pl.ds(s * ROWS_PER_TILE + off, sz)])
        off += sz
    pltpu.sync_copy(onesz_hbm.at[0], ones_v)
    plsc.subcore_barrier()

    pltpu.sync_copy(dst_hbm.at[base], dst_v.at[0])
    pltpu.make_async_copy(dst_hbm.at[base + 1], dst_v.at[1], sem_idx).start()

    def step(j, _):
        b = lax.rem(j, 2)
        pltpu.sync_copy(ones_v, acc.at[dst_v.at[b]], add=True)

        @pl.when(j + 1 < NCHUNK)
        def _():
            pltpu.make_async_copy(dst_hbm.at[base + j + 1],
                                  dst_v.at[1 - b], sem_idx).wait()

        @pl.when(j + 2 < NCHUNK)
        def _():
            pltpu.make_async_copy(dst_hbm.at[base + j + 2],
                                  dst_v.at[b], sem_idx).start()
        return 0
    lax.fori_loop(0, NCHUNK, step, 0)
    plsc.subcore_barrier()

    pltpu.sync_copy(acc.at[pl.ds(s * ROWS_PER_TILE, ROWS_PER_TILE)],
                    out_hbm.at[c, pl.ds(s * ROWS_PER_TILE, ROWS_PER_TILE)])


@functools.partial(
    pl.kernel,
    out_type=jax.ShapeDtypeStruct((NC, NPAD, D), jnp.float32),
    mesh=_MESH,
    scratch_types=[
        pltpu.VMEM((3, CH), jnp.int32),
        pltpu.VMEM((3, CH), jnp.int32),
        pltpu.VMEM((3, CH, D), jnp.float32),
        pltpu.VMEM_SHARED((NPAD, D), jnp.float32),
        pltpu.SemaphoreType.DMA,
        pltpu.SemaphoreType.DMA,
    ],
)
def _scatter_kernel(g_hbm, src_hbm, dst_hbm, out_hbm,
                    src_v, dst_v, rows_v, acc, sem, sem_idx):
    c = lax.axis_index("c")
    s = lax.axis_index("s")
    base = s * CPP + c * CN0
    cnt = jnp.where(c == 0, CN0, CN1)

    zero16 = jnp.zeros((16,), jnp.float32)

    def fill(r, _):
        def fcol(k, _):
            rows_v[0, r, pl.ds(k * 16, 16)] = zero16
            return 0
        return lax.fori_loop(0, D // 16, fcol, 0)
    lax.fori_loop(0, CH, fill, 0)

    off = 0
    for sz in _ZSIZES:
        pltpu.sync_copy(rows_v.at[0, pl.ds(0, sz)],
                        acc.at[pl.ds(s * ROWS_PER_TILE + off, sz)])
        off += sz
    plsc.subcore_barrier()

    @pl.when(0 < cnt)
    def _():
        pltpu.sync_copy(src_hbm.at[base], src_v.at[0])
        pltpu.sync_copy(dst_hbm.at[base], dst_v.at[0])
        pltpu.make_async_copy(g_hbm.at[src_v.at[0]], rows_v.at[0],
                              sem).start()

    @pl.when(1 < cnt)
    def _():
        pltpu.sync_copy(src_hbm.at[base + 1], src_v.at[1])
        pltpu.sync_copy(dst_hbm.at[base + 1], dst_v.at[1])
        pltpu.make_async_copy(g_hbm.at[src_v.at[1]], rows_v.at[1],
                              sem).start()

    @pl.when(2 < cnt)
    def _():
        pltpu.make_async_copy(src_hbm.at[base + 2], src_v.at[2],
                              sem_idx).start()
        pltpu.make_async_copy(dst_hbm.at[base + 2], dst_v.at[2],
                              sem_idx).start()

    def step(j, _):
        b = lax.rem(j, 3)
        b2 = lax.rem(j + 2, 3)
        pltpu.make_async_copy(g_hbm.at[src_v.at[b]], rows_v.at[b], sem).wait()

        @pl.when(j + 2 < cnt)
        def _():
            pltpu.make_async_copy(src_hbm.at[base + j + 2],
                                  src_v.at[b2], sem_idx).wait()
            pltpu.make_async_copy(dst_hbm.at[base + j + 2],
                                  dst_v.at[b2], sem_idx).wait()
            pltpu.make_async_copy(g_hbm.at[src_v.at[b2]],
                                  rows_v.at[b2], sem).start()

        pltpu.sync_copy(rows_v.at[b], acc.at[dst_v.at[b]], add=True)

        @pl.when(j + 3 < cnt)
        def _():
            pltpu.make_async_copy(src_hbm.at[base + j + 3],
                                  src_v.at[b], sem_idx).start()
            pltpu.make_async_copy(dst_hbm.at[base + j + 3],
                                  dst_v.at[b], sem_idx).start()
        return 0
    lax.fori_loop(0, cnt, step, 0)
    plsc.subcore_barrier()

    pltpu.sync_copy(acc.at[pl.ds(s * ROWS_PER_TILE, ROWS_PER_TILE)],
                    out_hbm.at[c, pl.ds(s * ROWS_PER_TILE, ROWS_PER_TILE)])


_BLK = 632


def _scale0(deg_part, x):
    def body(dp, xr, nrm_ref, g1_ref):
        a = dp[...]
        d = a[0, :, 0:1] + a[1, :, 0:1]
        nrm = lax.rsqrt(jnp.maximum(d, 1.0))
        nrm_ref[...] = nrm
        g1_ref[...] = xr[...] * nrm

    return pl.pallas_call(
        body,
        grid=(NPAD // _BLK,),
        in_specs=[pl.BlockSpec((2, _BLK, D), lambda i: (0, i, 0)),
                  pl.BlockSpec((_BLK, D), lambda i: (i, 0))],
        out_specs=(pl.BlockSpec((_BLK, 1), lambda i: (i, 0)),
                   pl.BlockSpec((_BLK, D), lambda i: (i, 0))),
        out_shape=(jax.ShapeDtypeStruct((NPAD, 1), jnp.float32),
                   jax.ShapeDtypeStruct((NPAD, D), jnp.float32)),
    )(deg_part, x)


def _combine1(p, norm):
    def body(pr, nr, h1_ref, g2_ref):
        a = pr[...]
        nrm = nr[...]
        h1 = (a[0] + a[1]) * nrm
        h1_ref[...] = h1
        g2_ref[...] = h1 * nrm

    return pl.pallas_call(
        body,
        grid=(NPAD // _BLK,),
        in_specs=[pl.BlockSpec((2, _BLK, D), lambda i: (0, i, 0)),
                  pl.BlockSpec((_BLK, 1), lambda i: (i, 0))],
        out_specs=(pl.BlockSpec((_BLK, D), lambda i: (i, 0)),
                   pl.BlockSpec((_BLK, D), lambda i: (i, 0))),
        out_shape=(jax.ShapeDtypeStruct((NPAD, D), jnp.float32),
                   jax.ShapeDtypeStruct((NPAD, D), jnp.float32)),
    )(p, norm)


def _final(q, norm, x, h1):
    def body(qr, nr, xr, h1r, o_ref):
        a = qr[...]
        h2 = (a[0] + a[1]) * nr[...]
        o_ref[...] = (xr[...] + h1r[...] + h2) * (1.0 / 3.0)

    return pl.pallas_call(
        body,
        grid=(NPAD // _BLK,),
        in_specs=[pl.BlockSpec((2, _BLK, D), lambda i: (0, i, 0)),
                  pl.BlockSpec((_BLK, 1), lambda i: (i, 0)),
                  pl.BlockSpec((_BLK, D), lambda i: (i, 0)),
                  pl.BlockSpec((_BLK, D), lambda i: (i, 0))],
        out_specs=pl.BlockSpec((_BLK, D), lambda i: (i, 0)),
        out_shape=jax.ShapeDtypeStruct((NPAD, D), jnp.float32),
    )(q, norm, x, h1)


def kernel(features, edge_index):
    ei = edge_index.astype(jnp.int32)
    trash = jnp.full((EPAD - N_EDGES,), NPAD - 1, jnp.int32)
    src = jnp.concatenate([ei[0], trash]).reshape(NCHUNKS_TOTAL, CH)
    dst = jnp.concatenate([ei[1], trash]).reshape(NCHUNKS_TOTAL, CH)
    x = jnp.pad(features, ((0, NPAD - N_NODES), (0, 0)))

    onesz = jnp.stack([jnp.ones((CH, D), jnp.float32),
                       jnp.zeros((CH, D), jnp.float32)])
    deg_part = _deg_kernel(onesz, dst)
    norm, g1 = _scale0(deg_part, x)
    p = _scatter_kernel(g1, src, dst)
    h1, g2 = _combine1(p, norm)
    q = _scatter_kernel(g2, src, dst)
    out = _final(q, norm, x, h1)
    return out[:N_NODES]


# speedup vs baseline: 1.4982x; 1.0462x over previous
import functools

import jax
import jax.numpy as jnp
from jax import lax
from jax.experimental import pallas as pl
from jax.experimental.pallas import tpu as pltpu
from jax.experimental.pallas import tpu_sc as plsc

N_NODES = 10000
D = 128
N_EDGES = 320000

NC = 2
NS = 16
NW = NC * NS

NPAD = 10112
ROWS_PER_TILE = NPAD // NS

CH = 128
NCHUNK = 79
EW = CH * NCHUNK
EPAD = EW * NW
NCHUNKS_TOTAL = EPAD // CH
CPP = 2 * NCHUNK
CN0 = 126
CN1 = CPP - CN0
_ZSIZES = [CH] * (ROWS_PER_TILE // CH) + (
    [ROWS_PER_TILE % CH] if ROWS_PER_TILE % CH else [])

_MESH = plsc.VectorSubcoreMesh(core_axis_name="c", subcore_axis_name="s")


@functools.partial(
    pl.kernel,
    out_type=jax.ShapeDtypeStruct((NC, NPAD, D), jnp.float32),
    mesh=_MESH,
    scratch_types=[
        pltpu.VMEM((2, CH), jnp.int32),
        pltpu.VMEM((CH, D), jnp.float32),
        pltpu.VMEM_SHARED((NPAD, D), jnp.float32),
        pltpu.SemaphoreType.DMA,
    ],
)
def _deg_kernel(onesz_hbm, dst_hbm, out_hbm, dst_v, ones_v, acc, sem_idx):
    c = lax.axis_index("c")
    s = lax.axis_index("s")
    base = (s * NC + c) * NCHUNK

    pltpu.sync_copy(onesz_hbm.at[1], ones_v)
    off = 0
    for sz in _ZSIZES:
        pltpu.sync_copy(ones_v.at[pl.ds(0, sz)],
                        acc.at[pl.ds(s * ROWS_PER_TILE + off, sz)])
        off += sz
    pltpu.sync_copy(onesz_hbm.at[0], ones_v)
    plsc.subcore_barrier()

    pltpu.sync_copy(dst_hbm.at[base], dst_v.at[0])
    pltpu.make_async_copy(dst_hbm.at[base + 1], dst_v.at[1], sem_idx).start()

    def step(j, _):
        b = lax.rem(j, 2)
        pltpu.sync_copy(ones_v, acc.at[dst_v.at[b]], add=True)

        @pl.when(j + 1 < NCHUNK)
        def _():
            pltpu.make_async_copy(dst_hbm.at[base + j + 1],
                                  dst_v.at[1 - b], sem_idx).wait()

        @pl.when(j + 2 < NCHUNK)
        def _():
            pltpu.make_async_copy(dst_hbm.at[base + j + 2],
                                  dst_v.at[b], sem_idx).start()
        return 0
    lax.fori_loop(0, NCHUNK, step, 0)
    plsc.subcore_barrier()

    pltpu.sync_copy(acc.at[pl.ds(s * ROWS_PER_TILE, ROWS_PER_TILE)],
                    out_hbm.at[c, pl.ds(s * ROWS_PER_TILE, ROWS_PER_TILE)])


@functools.partial(
    pl.kernel,
    out_type=jax.ShapeDtypeStruct((NC, NPAD, D), jnp.float32),
    mesh=_MESH,
    scratch_types=[
        pltpu.VMEM((3, CH), jnp.int32),
        pltpu.VMEM((3, CH), jnp.int32),
        pltpu.VMEM((3, CH, D), jnp.float32),
        pltpu.VMEM_SHARED((NPAD, D), jnp.float32),
        pltpu.SemaphoreType.DMA,
        pltpu.SemaphoreType.DMA,
    ],
)
def _scatter_kernel(g_hbm, src_hbm, dst_hbm, out_hbm,
                    src_v, dst_v, rows_v, acc, sem, sem_idx):
    c = lax.axis_index("c")
    s = lax.axis_index("s")
    base = s * CPP + c * CN0
    cnt = jnp.where(c == 0, CN0, CN1)

    zero16 = jnp.zeros((16,), jnp.float32)

    def fill(r, _):
        def fcol(k, _):
            rows_v[0, r, pl.ds(k * 16, 16)] = zero16
            return 0
        return lax.fori_loop(0, D // 16, fcol, 0)
    lax.fori_loop(0, CH, fill, 0)

    off = 0
    for sz in _ZSIZES:
        pltpu.sync_copy(rows_v.at[0, pl.ds(0, sz)],
                        acc.at[pl.ds(s * ROWS_PER_TILE + off, sz)])
        off += sz
    plsc.subcore_barrier()

    @pl.when(0 < cnt)
    def _():
        pltpu.sync_copy(src_hbm.at[base], src_v.at[0])
        pltpu.sync_copy(dst_hbm.at[base], dst_v.at[0])
        pltpu.make_async_copy(g_hbm.at[src_v.at[0]], rows_v.at[0],
                              sem).start()

    @pl.when(1 < cnt)
    def _():
        pltpu.sync_copy(src_hbm.at[base + 1], src_v.at[1])
        pltpu.sync_copy(dst_hbm.at[base + 1], dst_v.at[1])
        pltpu.make_async_copy(g_hbm.at[src_v.at[1]], rows_v.at[1],
                              sem).start()

    @pl.when(2 < cnt)
    def _():
        pltpu.make_async_copy(src_hbm.at[base + 2], src_v.at[2],
                              sem_idx).start()
        pltpu.make_async_copy(dst_hbm.at[base + 2], dst_v.at[2],
                              sem_idx).start()

    def step(j, _):
        b = lax.rem(j, 3)
        b2 = lax.rem(j + 2, 3)
        pltpu.make_async_copy(g_hbm.at[src_v.at[b]], rows_v.at[b], sem).wait()

        @pl.when(j + 2 < cnt)
        def _():
            pltpu.make_async_copy(src_hbm.at[base + j + 2],
                                  src_v.at[b2], sem_idx).wait()
            pltpu.make_async_copy(dst_hbm.at[base + j + 2],
                                  dst_v.at[b2], sem_idx).wait()
            pltpu.make_async_copy(g_hbm.at[src_v.at[b2]],
                                  rows_v.at[b2], sem).start()

        pltpu.sync_copy(rows_v.at[b], acc.at[dst_v.at[b]], add=True)

        @pl.when(j + 3 < cnt)
        def _():
            pltpu.make_async_copy(src_hbm.at[base + j + 3],
                                  src_v.at[b], sem_idx).start()
            pltpu.make_async_copy(dst_hbm.at[base + j + 3],
                                  dst_v.at[b], sem_idx).start()
        return 0
    lax.fori_loop(0, cnt, step, 0)
    plsc.subcore_barrier()

    pltpu.sync_copy(acc.at[pl.ds(s * ROWS_PER_TILE, ROWS_PER_TILE)],
                    out_hbm.at[c, pl.ds(s * ROWS_PER_TILE, ROWS_PER_TILE)])


_BLK = 632


def _scale0(deg_part, x):
    def body(dp, xr, nrm_ref, g1_ref):
        a = dp[...]
        d = a[0, :, 0:1] + a[1, :, 0:1]
        nrm = lax.rsqrt(jnp.maximum(d, 1.0))
        nrm_ref[...] = nrm
        g1_ref[...] = xr[...] * nrm

    return pl.pallas_call(
        body,
        grid=(NPAD // _BLK,),
        in_specs=[pl.BlockSpec((2, _BLK, D), lambda i: (0, i, 0)),
                  pl.BlockSpec((_BLK, D), lambda i: (i, 0))],
        out_specs=(pl.BlockSpec((_BLK, 1), lambda i: (i, 0)),
                   pl.BlockSpec((_BLK, D), lambda i: (i, 0))),
        out_shape=(jax.ShapeDtypeStruct((NPAD, 1), jnp.float32),
                   jax.ShapeDtypeStruct((NPAD, D), jnp.float32)),
    )(deg_part, x)


def _combine1(p, norm):
    def body(pr, nr, h1_ref, g2_ref):
        a = pr[...]
        nrm = nr[...]
        h1 = (a[0] + a[1]) * nrm
        h1_ref[...] = h1
        g2_ref[...] = h1 * nrm

    return pl.pallas_call(
        body,
        grid=(NPAD // _BLK,),
        in_specs=[pl.BlockSpec((2, _BLK, D), lambda i: (0, i, 0)),
                  pl.BlockSpec((_BLK, 1), lambda i: (i, 0))],
        out_specs=(pl.BlockSpec((_BLK, D), lambda i: (i, 0)),
                   pl.BlockSpec((_BLK, D), lambda i: (i, 0))),
        out_shape=(jax.ShapeDtypeStruct((NPAD, D), jnp.float32),
                   jax.ShapeDtypeStruct((NPAD, D), jnp.float32)),
    )(p, norm)


def _final(q, norm, x, h1):
    def body(qr, nr, xr, h1r, o_ref):
        a = qr[...]
        h2 = (a[0] + a[1]) * nr[...]
        o_ref[...] = (xr[...] + h1r[...] + h2) * (1.0 / 3.0)

    return pl.pallas_call(
        body,
        grid=(NPAD // _BLK,),
        in_specs=[pl.BlockSpec((2, _BLK, D), lambda i: (0, i, 0)),
                  pl.BlockSpec((_BLK, 1), lambda i: (i, 0)),
                  pl.BlockSpec((_BLK, D), lambda i: (i, 0)),
                  pl.BlockSpec((_BLK, D), lambda i: (i, 0))],
        out_specs=pl.BlockSpec((_BLK, D), lambda i: (i, 0)),
        out_shape=jax.ShapeDtypeStruct((NPAD, D), jnp.float32),
    )(q, norm, x, h1)


def kernel(features, edge_index):
    ei = edge_index.astype(jnp.int32)
    trash = jnp.full((EPAD - N_EDGES,), NPAD - 1, jnp.int32)
    src = jnp.concatenate([ei[0], trash]).reshape(NCHUNKS_TOTAL, CH)
    dst = jnp.concatenate([ei[1], trash]).reshape(NCHUNKS_TOTAL, CH)
    x = jnp.pad(features, ((0, NPAD - N_NODES), (0, 0)))

    onesz = jnp.stack([jnp.ones((CH, D), jnp.float32),
                       jnp.zeros((CH, D), jnp.float32)])
    deg_part = _deg_kernel(onesz, dst)
    norm, g1 = _scale0(deg_part, x)
    p = _scatter_kernel(g1, src, dst)
    h1, g2 = _combine1(p, norm)
    q = _scatter_kernel(g2, src, dst)
    out = _final(q, norm, x, h1)
    return out[:N_NODES]


# speedup vs baseline: 1.5083x; 1.0067x over previous
import functools

import jax
import jax.numpy as jnp
from jax import lax
from jax.experimental import pallas as pl
from jax.experimental.pallas import tpu as pltpu
from jax.experimental.pallas import tpu_sc as plsc

N_NODES = 10000
D = 128
N_EDGES = 320000

NC = 2
NS = 16
NW = NC * NS

NPAD = 10112
ROWS_PER_TILE = NPAD // NS

CH = 128
NCHUNK = 79
EW = CH * NCHUNK
EPAD = EW * NW
NCHUNKS_TOTAL = EPAD // CH
CPP = 2 * NCHUNK
CN0 = 138
CN1 = CPP - CN0
_ZSIZES = [CH] * (ROWS_PER_TILE // CH) + (
    [ROWS_PER_TILE % CH] if ROWS_PER_TILE % CH else [])

_MESH = plsc.VectorSubcoreMesh(core_axis_name="c", subcore_axis_name="s")


@functools.partial(
    pl.kernel,
    out_type=jax.ShapeDtypeStruct((NC, NPAD, D), jnp.float32),
    mesh=_MESH,
    scratch_types=[
        pltpu.VMEM((2, CH), jnp.int32),
        pltpu.VMEM((CH, D), jnp.float32),
        pltpu.VMEM_SHARED((NPAD, D), jnp.float32),
        pltpu.SemaphoreType.DMA,
    ],
)
def _deg_kernel(onesz_hbm, dst_hbm, out_hbm, dst_v, ones_v, acc, sem_idx):
    c = lax.axis_index("c")
    s = lax.axis_index("s")
    base = (s * NC + c) * NCHUNK

    pltpu.sync_copy(onesz_hbm.at[1], ones_v)
    off = 0
    for sz in _ZSIZES:
        pltpu.sync_copy(ones_v.at[pl.ds(0, sz)],
                        acc.at[pl.ds(s * ROWS_PER_TILE + off, sz)])
        off += sz
    pltpu.sync_copy(onesz_hbm.at[0], ones_v)
    plsc.subcore_barrier()

    pltpu.sync_copy(dst_hbm.at[base], dst_v.at[0])
    pltpu.make_async_copy(dst_hbm.at[base + 1], dst_v.at[1], sem_idx).start()

    def step(j, _):
        b = lax.rem(j, 2)
        pltpu.sync_copy(ones_v, acc.at[dst_v.at[b]], add=True)

        @pl.when(j + 1 < NCHUNK)
        def _():
            pltpu.make_async_copy(dst_hbm.at[base + j + 1],
                                  dst_v.at[1 - b], sem_idx).wait()

        @pl.when(j + 2 < NCHUNK)
        def _():
            pltpu.make_async_copy(dst_hbm.at[base + j + 2],
                                  dst_v.at[b], sem_idx).start()
        return 0
    lax.fori_loop(0, NCHUNK, step, 0)
    plsc.subcore_barrier()

    pltpu.sync_copy(acc.at[pl.ds(s * ROWS_PER_TILE, ROWS_PER_TILE)],
                    out_hbm.at[c, pl.ds(s * ROWS_PER_TILE, ROWS_PER_TILE)])


@functools.partial(
    pl.kernel,
    out_type=jax.ShapeDtypeStruct((NC, NPAD, D), jnp.float32),
    mesh=_MESH,
    scratch_types=[
        pltpu.VMEM((3, CH), jnp.int32),
        pltpu.VMEM((3, CH), jnp.int32),
        pltpu.VMEM((3, CH, D), jnp.float32),
        pltpu.VMEM_SHARED((NPAD, D), jnp.float32),
        pltpu.SemaphoreType.DMA,
        pltpu.SemaphoreType.DMA,
    ],
)
def _scatter_kernel(g_hbm, src_hbm, dst_hbm, out_hbm,
                    src_v, dst_v, rows_v, acc, sem, sem_idx):
    c = lax.axis_index("c")
    s = lax.axis_index("s")
    base = s * CPP + c * CN0
    cnt = jnp.where(c == 0, CN0, CN1)

    zero16 = jnp.zeros((16,), jnp.float32)

    def fill(r, _):
        def fcol(k, _):
            rows_v[0, r, pl.ds(k * 16, 16)] = zero16
            return 0
        return lax.fori_loop(0, D // 16, fcol, 0)
    lax.fori_loop(0, CH, fill, 0)

    off = 0
    for sz in _ZSIZES:
        pltpu.sync_copy(rows_v.at[0, pl.ds(0, sz)],
                        acc.at[pl.ds(s * ROWS_PER_TILE + off, sz)])
        off += sz
    plsc.subcore_barrier()

    @pl.when(0 < cnt)
    def _():
        pltpu.sync_copy(src_hbm.at[base], src_v.at[0])
        pltpu.sync_copy(dst_hbm.at[base], dst_v.at[0])
        pltpu.make_async_copy(g_hbm.at[src_v.at[0]], rows_v.at[0],
                              sem).start()

    @pl.when(1 < cnt)
    def _():
        pltpu.sync_copy(src_hbm.at[base + 1], src_v.at[1])
        pltpu.sync_copy(dst_hbm.at[base + 1], dst_v.at[1])
        pltpu.make_async_copy(g_hbm.at[src_v.at[1]], rows_v.at[1],
                              sem).start()

    @pl.when(2 < cnt)
    def _():
        pltpu.make_async_copy(src_hbm.at[base + 2], src_v.at[2],
                              sem_idx).start()
        pltpu.make_async_copy(dst_hbm.at[base + 2], dst_v.at[2],
                              sem_idx).start()

    def step(j, _):
        b = lax.rem(j, 3)
        b2 = lax.rem(j + 2, 3)
        pltpu.make_async_copy(g_hbm.at[src_v.at[b]], rows_v.at[b], sem).wait()

        @pl.when(j + 2 < cnt)
        def _():
            pltpu.make_async_copy(src_hbm.at[base + j + 2],
                                  src_v.at[b2], sem_idx).wait()
            pltpu.make_async_copy(dst_hbm.at[base + j + 2],
                                  dst_v.at[b2], sem_idx).wait()
            pltpu.make_async_copy(g_hbm.at[src_v.at[b2]],
                                  rows_v.at[b2], sem).start()

        pltpu.sync_copy(rows_v.at[b], acc.at[dst_v.at[b]], add=True)

        @pl.when(j + 3 < cnt)
        def _():
            pltpu.make_async_copy(src_hbm.at[base + j + 3],
                                  src_v.at[b], sem_idx).start()
            pltpu.make_async_copy(dst_hbm.at[base + j + 3],
                                  dst_v.at[b], sem_idx).start()
        return 0
    lax.fori_loop(0, cnt, step, 0)
    plsc.subcore_barrier()

    pltpu.sync_copy(acc.at[pl.ds(s * ROWS_PER_TILE, ROWS_PER_TILE)],
                    out_hbm.at[c, pl.ds(s * ROWS_PER_TILE, ROWS_PER_TILE)])


_BLK = 632


def _scale0(deg_part, x):
    def body(dp, xr, nrm_ref, g1_ref):
        a = dp[...]
        d = a[0, :, 0:1] + a[1, :, 0:1]
        nrm = lax.rsqrt(jnp.maximum(d, 1.0))
        nrm_ref[...] = nrm
        g1_ref[...] = xr[...] * nrm

    return pl.pallas_call(
        body,
        grid=(NPAD // _BLK,),
        in_specs=[pl.BlockSpec((2, _BLK, D), lambda i: (0, i, 0)),
                  pl.BlockSpec((_BLK, D), lambda i: (i, 0))],
        out_specs=(pl.BlockSpec((_BLK, 1), lambda i: (i, 0)),
                   pl.BlockSpec((_BLK, D), lambda i: (i, 0))),
        out_shape=(jax.ShapeDtypeStruct((NPAD, 1), jnp.float32),
                   jax.ShapeDtypeStruct((NPAD, D), jnp.float32)),
    )(deg_part, x)


def _combine1(p, norm):
    def body(pr, nr, h1_ref, g2_ref):
        a = pr[...]
        nrm = nr[...]
        h1 = (a[0] + a[1]) * nrm
        h1_ref[...] = h1
        g2_ref[...] = h1 * nrm

    return pl.pallas_call(
        body,
        grid=(NPAD // _BLK,),
        in_specs=[pl.BlockSpec((2, _BLK, D), lambda i: (0, i, 0)),
                  pl.BlockSpec((_BLK, 1), lambda i: (i, 0))],
        out_specs=(pl.BlockSpec((_BLK, D), lambda i: (i, 0)),
                   pl.BlockSpec((_BLK, D), lambda i: (i, 0))),
        out_shape=(jax.ShapeDtypeStruct((NPAD, D), jnp.float32),
                   jax.ShapeDtypeStruct((NPAD, D), jnp.float32)),
    )(p, norm)


def _final(q, norm, x, h1):
    def body(qr, nr, xr, h1r, o_ref):
        a = qr[...]
        h2 = (a[0] + a[1]) * nr[...]
        o_ref[...] = (xr[...] + h1r[...] + h2) * (1.0 / 3.0)

    return pl.pallas_call(
        body,
        grid=(NPAD // _BLK,),
        in_specs=[pl.BlockSpec((2, _BLK, D), lambda i: (0, i, 0)),
                  pl.BlockSpec((_BLK, 1), lambda i: (i, 0)),
                  pl.BlockSpec((_BLK, D), lambda i: (i, 0)),
                  pl.BlockSpec((_BLK, D), lambda i: (i, 0))],
        out_specs=pl.BlockSpec((_BLK, D), lambda i: (i, 0)),
        out_shape=jax.ShapeDtypeStruct((NPAD, D), jnp.float32),
    )(q, norm, x, h1)


def kernel(features, edge_index):
    ei = edge_index.astype(jnp.int32)
    trash = jnp.full((EPAD - N_EDGES,), NPAD - 1, jnp.int32)
    src = jnp.concatenate([ei[0], trash]).reshape(NCHUNKS_TOTAL, CH)
    dst = jnp.concatenate([ei[1], trash]).reshape(NCHUNKS_TOTAL, CH)
    x = jnp.pad(features, ((0, NPAD - N_NODES), (0, 0)))

    onesz = jnp.stack([jnp.ones((CH, D), jnp.float32),
                       jnp.zeros((CH, D), jnp.float32)])
    deg_part = _deg_kernel(onesz, dst)
    norm, g1 = _scale0(deg_part, x)
    p = _scatter_kernel(g1, src, dst)
    h1, g2 = _combine1(p, norm)
    q = _scatter_kernel(g2, src, dst)
    out = _final(q, norm, x, h1)
    return out[:N_NODES]
